# baseline (speedup 1.0000x reference)
import functools

import jax
import jax.numpy as jnp
from jax import lax
from jax.experimental import pallas as pl
from jax.experimental.pallas import tpu as pltpu
from jax.experimental.pallas import tpu_sc as plsc

N_NODES = 10000
N_EDGES = 320000
IN_CH = 128
HID = 256
HALF = HID // 2
NP = 10240

NC = 2
NS = 16
LANES = 16

CHUNK = 128
E_PAD = 327680
IDX_ROWS = E_PAD // CHUNK
TILE_IDX_ROWS = IDX_ROWS // NS
IDX_BLOCK = 40
DEG_IDX_ROWS = IDX_ROWS // NC // NS
ROWS_PER_TILE = NP // NS
DEG_PER_TILE = NP // NS

_MESH = plsc.VectorSubcoreMesh(
    core_axis_name="c", subcore_axis_name="s", num_cores=NC, num_subcores=NS
)


@functools.partial(
    pl.kernel,
    out_type=jax.ShapeDtypeStruct((NC, NP), jnp.float32),
    mesh=_MESH,
    scratch_types=[
        pltpu.VMEM((DEG_IDX_ROWS, CHUNK), jnp.int32),
        pltpu.VMEM((CHUNK,), jnp.float32),
        pltpu.VMEM((DEG_PER_TILE,), jnp.float32),
        pltpu.VMEM_SHARED((NP,), jnp.float32),
    ],
)
def _sc_degree(dst_hbm, ones_hbm, out_hbm, idx_v, ones_v, zero_v, acc_sh):
    c = lax.axis_index("c")
    s = lax.axis_index("s")
    zero = jnp.zeros((LANES,), jnp.float32)
    for j in range(DEG_PER_TILE // LANES):
        zero_v[pl.ds(j * LANES, LANES)] = zero
    pltpu.sync_copy(zero_v, acc_sh.at[pl.ds(s * DEG_PER_TILE, DEG_PER_TILE)])
    pltpu.sync_copy(ones_hbm, ones_v)
    ibase = c * (IDX_ROWS // NC) + s * DEG_IDX_ROWS
    pltpu.sync_copy(dst_hbm.at[pl.ds(ibase, DEG_IDX_ROWS)], idx_v)
    plsc.subcore_barrier()

    def body(i, carry):
        pltpu.sync_copy(ones_v, acc_sh.at[idx_v.at[i]], add=True)
        return carry

    lax.fori_loop(0, DEG_IDX_ROWS, body, 0)
    plsc.subcore_barrier()
    pltpu.sync_copy(
        acc_sh.at[pl.ds(s * DEG_PER_TILE, DEG_PER_TILE)],
        out_hbm.at[c, pl.ds(s * DEG_PER_TILE, DEG_PER_TILE)],
    )


@functools.partial(
    pl.kernel,
    out_type=jax.ShapeDtypeStruct((NC, NP, HALF), jnp.float32),
    mesh=_MESH,
    scratch_types=[
        pltpu.VMEM((IDX_BLOCK, CHUNK), jnp.int32),
        pltpu.VMEM((IDX_BLOCK, CHUNK), jnp.int32),
        pltpu.VMEM((CHUNK, HALF), jnp.float32),
        pltpu.VMEM((CHUNK, HALF), jnp.float32),
        pltpu.VMEM_SHARED((NP, HALF), jnp.float32),
        pltpu.SemaphoreType.DMA,
        pltpu.SemaphoreType.DMA,
    ],
)
def _sc_aggregate(hp_hbm, srcab_hbm, dst_hbm, zeros_hbm, out_hbm,
                  sidx_v, didx_v, rows0_v, rows1_v, acc_sh, sem0, sem1):
    c = lax.axis_index("c")
    s = lax.axis_index("s")
    rbase = s * ROWS_PER_TILE
    pltpu.sync_copy(
        zeros_hbm.at[pl.ds(rbase, ROWS_PER_TILE)],
        acc_sh.at[pl.ds(rbase, ROWS_PER_TILE)],
    )
    ibase = s * TILE_IDX_ROWS
    plsc.subcore_barrier()

    def block(b, carry):
        pltpu.sync_copy(
            srcab_hbm.at[c, pl.ds(ibase + b * IDX_BLOCK, IDX_BLOCK)], sidx_v
        )
        pltpu.sync_copy(
            dst_hbm.at[pl.ds(ibase + b * IDX_BLOCK, IDX_BLOCK)], didx_v
        )
        pltpu.async_copy(hp_hbm.at[sidx_v.at[0]], rows0_v, sem0)

        def pair(k, c2):
            i0 = 2 * k
            i1 = i0 + 1
            pltpu.make_async_copy(
                zeros_hbm.at[pl.ds(0, CHUNK)], rows0_v, sem0).wait()
            pltpu.async_copy(hp_hbm.at[sidx_v.at[i1]], rows1_v, sem1)
            pltpu.sync_copy(rows0_v, acc_sh.at[didx_v.at[i0]], add=True)
            pltpu.make_async_copy(
                zeros_hbm.at[pl.ds(0, CHUNK)], rows1_v, sem1).wait()

            @pl.when(k < IDX_BLOCK // 2 - 1)
            def _():
                pltpu.async_copy(hp_hbm.at[sidx_v.at[i0 + 2]], rows0_v, sem0)

            pltpu.sync_copy(rows1_v, acc_sh.at[didx_v.at[i1]], add=True)
            return c2

        lax.fori_loop(0, IDX_BLOCK // 2, pair, 0)
        return carry

    lax.fori_loop(0, TILE_IDX_ROWS // IDX_BLOCK, block, 0)
    plsc.subcore_barrier()
    pltpu.sync_copy(
        acc_sh.at[pl.ds(rbase, ROWS_PER_TILE)],
        out_hbm.at[c, pl.ds(rbase, ROWS_PER_TILE)],
    )


def _tc1_body(x_ref, w_ref, dis_ref, out_ref):
    h = jnp.dot(x_ref[...], w_ref[...], preferred_element_type=jnp.float32)
    hp = dis_ref[...] * h
    out_ref[0] = hp[:, :HALF]
    out_ref[1] = hp[:, HALF:]


def _tc_mid_body(aa_ref, ha_ref, dis_ref, b_ref, w_ref, out_ref):
    a = jnp.concatenate([aa_ref[0] + ha_ref[0], aa_ref[1] + ha_ref[1]], axis=1)
    d = dis_ref[...]
    h = jnp.maximum(d * a + b_ref[0:1, :], 0.0)
    o = d * jnp.dot(h, w_ref[...], preferred_element_type=jnp.float32)
    out_ref[0] = o[:, :HALF]
    out_ref[1] = o[:, HALF:]


def _tc_final_body(aa_ref, ha_ref, dis_ref, b_ref, wl_ref, bl_ref, out_ref):
    a = jnp.concatenate([aa_ref[0] + ha_ref[0], aa_ref[1] + ha_ref[1]], axis=1)
    d = dis_ref[...]
    h = jnp.maximum(d * a + b_ref[0:1, :], 0.0)
    z = jnp.dot(h, wl_ref[...], preferred_element_type=jnp.float32)
    z = z + bl_ref[0:1, 0:1]
    out_ref[...] = 1.0 / (1.0 + jnp.exp(-z))


_R = 1024
_G = NP // _R


def kernel(x, edge_index, W1, b1, W2, b2, Wl, bl):
    src = edge_index[0].astype(jnp.int32)
    dst = edge_index[1].astype(jnp.int32)

    pad_n = E_PAD - N_EDGES
    pad_src = jnp.arange(pad_n, dtype=jnp.int32) % N_NODES
    pad_dst = N_NODES + (jnp.arange(pad_n, dtype=jnp.int32) % (NP - N_NODES))
    src_p = jnp.concatenate([src, pad_src])
    dst_p = jnp.concatenate([dst, pad_dst])
    src_ab = jnp.stack([src_p, src_p + NP]).reshape(NC, IDX_ROWS, CHUNK)
    dst2d = dst_p.reshape(IDX_ROWS, CHUNK)

    xp = jnp.pad(x, ((0, NP - N_NODES), (0, 0)))
    ones_hbm = jnp.ones((CHUNK,), jnp.float32)

    partials = _sc_degree(dst2d, ones_hbm)
    deg = partials[0] + partials[1] + 1.0
    dis = lax.rsqrt(deg).reshape(NP, 1)

    zeros = jnp.zeros((NP, HALF), jnp.float32)
    b1b = jnp.broadcast_to(b1.reshape(1, HID), (8, HID))
    b2b = jnp.broadcast_to(b2.reshape(1, HID), (8, HID))
    wlp = jnp.pad(Wl, ((0, 0), (0, HALF - 1)))
    blb = jnp.broadcast_to(bl.reshape(1, 1), (8, HALF))

    hp1 = pl.pallas_call(
        _tc1_body,
        grid=(_G,),
        in_specs=[
            pl.BlockSpec((_R, IN_CH), lambda i: (i, 0)),
            pl.BlockSpec((IN_CH, HID), lambda i: (0, 0)),
            pl.BlockSpec((_R, 1), lambda i: (i, 0)),
        ],
        out_specs=pl.BlockSpec((2, _R, HALF), lambda i: (0, i, 0)),
        out_shape=jax.ShapeDtypeStruct((2, NP, HALF), jnp.float32),
    )(xp, W1, dis)

    agg1 = _sc_aggregate(hp1.reshape(2 * NP, HALF), src_ab, dst2d, zeros)

    hp2 = pl.pallas_call(
        _tc_mid_body,
        grid=(_G,),
        in_specs=[
            pl.BlockSpec((2, _R, HALF), lambda i: (0, i, 0)),
            pl.BlockSpec((2, _R, HALF), lambda i: (0, i, 0)),
            pl.BlockSpec((_R, 1), lambda i: (i, 0)),
            pl.BlockSpec((8, HID), lambda i: (0, 0)),
            pl.BlockSpec((HID, HID), lambda i: (0, 0)),
        ],
        out_specs=pl.BlockSpec((2, _R, HALF), lambda i: (0, i, 0)),
        out_shape=jax.ShapeDtypeStruct((2, NP, HALF), jnp.float32),
    )(agg1, hp1, dis, b1b, W2)

    agg2 = _sc_aggregate(hp2.reshape(2 * NP, HALF), src_ab, dst2d, zeros)

    o = pl.pallas_call(
        _tc_final_body,
        grid=(_G,),
        in_specs=[
            pl.BlockSpec((2, _R, HALF), lambda i: (0, i, 0)),
            pl.BlockSpec((2, _R, HALF), lambda i: (0, i, 0)),
            pl.BlockSpec((_R, 1), lambda i: (i, 0)),
            pl.BlockSpec((8, HID), lambda i: (0, 0)),
            pl.BlockSpec((HID, HALF), lambda i: (0, 0)),
            pl.BlockSpec((8, HALF), lambda i: (0, 0)),
        ],
        out_specs=pl.BlockSpec((_R, HALF), lambda i: (i, 0)),
        out_shape=jax.ShapeDtypeStruct((NP, HALF), jnp.float32),
    )(agg2, hp2, dis, b2b, wlp, blb)

    return o[:N_NODES, 0]


# speedup vs baseline: 1.0154x; 1.0154x over previous
import functools

import jax
import jax.numpy as jnp
from jax import lax
from jax.experimental import pallas as pl
from jax.experimental.pallas import tpu as pltpu
from jax.experimental.pallas import tpu_sc as plsc

N_NODES = 10000
N_EDGES = 320000
IN_CH = 128
HID = 256
HALF = HID // 2
NP = 10240

NC = 2
NS = 16
LANES = 16

CHUNK = 128
E_PAD = 327680
IDX_ROWS = E_PAD // CHUNK
TILE_IDX_ROWS = IDX_ROWS // NS
IDX_BLOCK = 40
DEG_IDX_ROWS = IDX_ROWS // NC // NS
ROWS_PER_TILE = NP // NS
DEG_PER_TILE = NP // NS

_MESH = plsc.VectorSubcoreMesh(
    core_axis_name="c", subcore_axis_name="s", num_cores=NC, num_subcores=NS
)


@functools.partial(
    pl.kernel,
    out_type=jax.ShapeDtypeStruct((NC, NP), jnp.float32),
    mesh=_MESH,
    scratch_types=[
        pltpu.VMEM((DEG_IDX_ROWS, CHUNK), jnp.int32),
        pltpu.VMEM((CHUNK,), jnp.float32),
        pltpu.VMEM((DEG_PER_TILE,), jnp.float32),
        pltpu.VMEM_SHARED((NP,), jnp.float32),
    ],
)
def _sc_degree(dst_hbm, ones_hbm, out_hbm, idx_v, ones_v, zero_v, acc_sh):
    c = lax.axis_index("c")
    s = lax.axis_index("s")
    zero = jnp.zeros((LANES,), jnp.float32)
    for j in range(DEG_PER_TILE // LANES):
        zero_v[pl.ds(j * LANES, LANES)] = zero
    pltpu.sync_copy(zero_v, acc_sh.at[pl.ds(s * DEG_PER_TILE, DEG_PER_TILE)])
    pltpu.sync_copy(ones_hbm, ones_v)
    ibase = c * (IDX_ROWS // NC) + s * DEG_IDX_ROWS
    pltpu.sync_copy(dst_hbm.at[pl.ds(ibase, DEG_IDX_ROWS)], idx_v)
    plsc.subcore_barrier()

    def body(i, carry):
        pltpu.sync_copy(ones_v, acc_sh.at[idx_v.at[i]], add=True)
        return carry

    lax.fori_loop(0, DEG_IDX_ROWS, body, 0)
    plsc.subcore_barrier()
    pltpu.sync_copy(
        acc_sh.at[pl.ds(s * DEG_PER_TILE, DEG_PER_TILE)],
        out_hbm.at[c, pl.ds(s * DEG_PER_TILE, DEG_PER_TILE)],
    )


@functools.partial(
    pl.kernel,
    out_type=jax.ShapeDtypeStruct((NC, NP, HALF), jnp.float32),
    mesh=_MESH,
    scratch_types=[
        pltpu.VMEM((IDX_BLOCK, CHUNK), jnp.int32),
        pltpu.VMEM((IDX_BLOCK, CHUNK), jnp.int32),
        pltpu.VMEM((CHUNK, HALF), jnp.float32),
        pltpu.VMEM((CHUNK, HALF), jnp.float32),
        pltpu.VMEM_SHARED((NP, HALF), jnp.float32),
        pltpu.SemaphoreType.DMA,
        pltpu.SemaphoreType.DMA,
    ],
)
def _sc_aggregate(hp_hbm, srcab_hbm, dst_hbm, zeros_hbm, out_hbm,
                  sidx_v, didx_v, rows0_v, rows1_v, acc_sh, sem0, sem1):
    c = lax.axis_index("c")
    s = lax.axis_index("s")
    rbase = s * ROWS_PER_TILE
    pltpu.sync_copy(
        zeros_hbm.at[pl.ds(rbase, ROWS_PER_TILE)],
        acc_sh.at[pl.ds(rbase, ROWS_PER_TILE)],
    )
    ibase = s * TILE_IDX_ROWS
    plsc.subcore_barrier()

    def block(b, carry):
        pltpu.sync_copy(
            srcab_hbm.at[c, pl.ds(ibase + b * IDX_BLOCK, IDX_BLOCK)], sidx_v
        )
        pltpu.sync_copy(
            dst_hbm.at[pl.ds(ibase + b * IDX_BLOCK, IDX_BLOCK)], didx_v
        )
        pltpu.async_copy(hp_hbm.at[sidx_v.at[0]], rows0_v, sem0)

        def pair(k, c2):
            i0 = 2 * k
            i1 = i0 + 1
            pltpu.make_async_copy(
                zeros_hbm.at[pl.ds(0, CHUNK)], rows0_v, sem0).wait()
            pltpu.async_copy(hp_hbm.at[sidx_v.at[i1]], rows1_v, sem1)
            pltpu.sync_copy(rows0_v, acc_sh.at[didx_v.at[i0]], add=True)
            pltpu.make_async_copy(
                zeros_hbm.at[pl.ds(0, CHUNK)], rows1_v, sem1).wait()

            @pl.when(k < IDX_BLOCK // 2 - 1)
            def _():
                pltpu.async_copy(hp_hbm.at[sidx_v.at[i0 + 2]], rows0_v, sem0)

            pltpu.sync_copy(rows1_v, acc_sh.at[didx_v.at[i1]], add=True)
            return c2

        lax.fori_loop(0, IDX_BLOCK // 2, pair, 0)
        return carry

    lax.fori_loop(0, TILE_IDX_ROWS // IDX_BLOCK, block, 0)
    plsc.subcore_barrier()
    pltpu.sync_copy(
        acc_sh.at[pl.ds(rbase, ROWS_PER_TILE)],
        out_hbm.at[c, pl.ds(rbase, ROWS_PER_TILE)],
    )


def _tc1_body(x_ref, w_ref, dis_ref, out_ref):
    h = jnp.dot(x_ref[...], w_ref[...], preferred_element_type=jnp.float32)
    hp = dis_ref[...] * h
    out_ref[0] = hp[:, :HALF]
    out_ref[1] = hp[:, HALF:]


def _tc_mid_body(aa_ref, ha_ref, dis_ref, b_ref, w_ref, out_ref):
    a = jnp.concatenate([aa_ref[0] + ha_ref[0], aa_ref[1] + ha_ref[1]], axis=1)
    d = dis_ref[...]
    h = jnp.maximum(d * a + b_ref[0:1, :], 0.0)
    o = d * jnp.dot(h, w_ref[...], preferred_element_type=jnp.float32)
    out_ref[0] = o[:, :HALF]
    out_ref[1] = o[:, HALF:]


def _tc_final_body(aa_ref, ha_ref, dis_ref, b_ref, wl_ref, bl_ref, out_ref):
    a = jnp.concatenate([aa_ref[0] + ha_ref[0], aa_ref[1] + ha_ref[1]], axis=1)
    d = dis_ref[...]
    h = jnp.maximum(d * a + b_ref[0:1, :], 0.0)
    z = jnp.dot(h, wl_ref[...], preferred_element_type=jnp.float32)
    z = z + bl_ref[0:1, 0:1]
    out_ref[...] = 1.0 / (1.0 + jnp.exp(-z))


_R = 2048
_G = NP // _R


def kernel(x, edge_index, W1, b1, W2, b2, Wl, bl):
    src = edge_index[0].astype(jnp.int32)
    dst = edge_index[1].astype(jnp.int32)

    pad_n = E_PAD - N_EDGES
    pad_src = jnp.arange(pad_n, dtype=jnp.int32) % N_NODES
    pad_dst = N_NODES + (jnp.arange(pad_n, dtype=jnp.int32) % (NP - N_NODES))
    src_p = jnp.concatenate([src, pad_src])
    dst_p = jnp.concatenate([dst, pad_dst])
    src_ab = jnp.stack([src_p, src_p + NP]).reshape(NC, IDX_ROWS, CHUNK)
    dst2d = dst_p.reshape(IDX_ROWS, CHUNK)

    xp = jnp.pad(x, ((0, NP - N_NODES), (0, 0)))
    ones_hbm = jnp.ones((CHUNK,), jnp.float32)

    partials = _sc_degree(dst2d, ones_hbm)
    deg = partials[0] + partials[1] + 1.0
    dis = lax.rsqrt(deg).reshape(NP, 1)

    zeros = jnp.zeros((NP, HALF), jnp.float32)
    b1b = jnp.broadcast_to(b1.reshape(1, HID), (8, HID))
    b2b = jnp.broadcast_to(b2.reshape(1, HID), (8, HID))
    wlp = jnp.pad(Wl, ((0, 0), (0, HALF - 1)))
    blb = jnp.broadcast_to(bl.reshape(1, 1), (8, HALF))

    hp1 = pl.pallas_call(
        _tc1_body,
        grid=(_G,),
        in_specs=[
            pl.BlockSpec((_R, IN_CH), lambda i: (i, 0)),
            pl.BlockSpec((IN_CH, HID), lambda i: (0, 0)),
            pl.BlockSpec((_R, 1), lambda i: (i, 0)),
        ],
        out_specs=pl.BlockSpec((2, _R, HALF), lambda i: (0, i, 0)),
        out_shape=jax.ShapeDtypeStruct((2, NP, HALF), jnp.float32),
    )(xp, W1, dis)

    agg1 = _sc_aggregate(hp1.reshape(2 * NP, HALF), src_ab, dst2d, zeros)

    hp2 = pl.pallas_call(
        _tc_mid_body,
        grid=(_G,),
        in_specs=[
            pl.BlockSpec((2, _R, HALF), lambda i: (0, i, 0)),
            pl.BlockSpec((2, _R, HALF), lambda i: (0, i, 0)),
            pl.BlockSpec((_R, 1), lambda i: (i, 0)),
            pl.BlockSpec((8, HID), lambda i: (0, 0)),
            pl.BlockSpec((HID, HID), lambda i: (0, 0)),
        ],
        out_specs=pl.BlockSpec((2, _R, HALF), lambda i: (0, i, 0)),
        out_shape=jax.ShapeDtypeStruct((2, NP, HALF), jnp.float32),
    )(agg1, hp1, dis, b1b, W2)

    agg2 = _sc_aggregate(hp2.reshape(2 * NP, HALF), src_ab, dst2d, zeros)

    o = pl.pallas_call(
        _tc_final_body,
        grid=(_G,),
        in_specs=[
            pl.BlockSpec((2, _R, HALF), lambda i: (0, i, 0)),
            pl.BlockSpec((2, _R, HALF), lambda i: (0, i, 0)),
            pl.BlockSpec((_R, 1), lambda i: (i, 0)),
            pl.BlockSpec((8, HID), lambda i: (0, 0)),
            pl.BlockSpec((HID, HALF), lambda i: (0, 0)),
            pl.BlockSpec((8, HALF), lambda i: (0, 0)),
        ],
        out_specs=pl.BlockSpec((_R, HALF), lambda i: (i, 0)),
        out_shape=jax.ShapeDtypeStruct((NP, HALF), jnp.float32),
    )(agg2, hp2, dis, b2b, wlp, blb)

    return o[:N_NODES, 0]


# speedup vs baseline: 1.0157x; 1.0003x over previous
import functools

import jax
import jax.numpy as jnp
from jax import lax
from jax.experimental import pallas as pl
from jax.experimental.pallas import tpu as pltpu
from jax.experimental.pallas import tpu_sc as plsc

N_NODES = 10000
N_EDGES = 320000
IN_CH = 128
HID = 256
HALF = HID // 2
NP = 10240

NC = 2
NS = 16
LANES = 16

CHUNK = 128
E_PAD = 327680
IDX_ROWS = E_PAD // CHUNK
TILE_IDX_ROWS = IDX_ROWS // NS
IDX_BLOCK = 40
DEG_IDX_ROWS = IDX_ROWS // NC // NS
ROWS_PER_TILE = NP // NS
DEG_PER_TILE = NP // NS

_MESH = plsc.VectorSubcoreMesh(
    core_axis_name="c", subcore_axis_name="s", num_cores=NC, num_subcores=NS
)


@functools.partial(
    pl.kernel,
    out_type=jax.ShapeDtypeStruct((NC, NP), jnp.float32),
    mesh=_MESH,
    scratch_types=[
        pltpu.VMEM((DEG_IDX_ROWS, CHUNK), jnp.int32),
        pltpu.VMEM((CHUNK,), jnp.float32),
        pltpu.VMEM((DEG_PER_TILE,), jnp.float32),
        pltpu.VMEM_SHARED((NP,), jnp.float32),
    ],
)
def _sc_degree(dst_hbm, ones_hbm, out_hbm, idx_v, ones_v, zero_v, acc_sh):
    c = lax.axis_index("c")
    s = lax.axis_index("s")
    zero = jnp.zeros((LANES,), jnp.float32)
    for j in range(DEG_PER_TILE // LANES):
        zero_v[pl.ds(j * LANES, LANES)] = zero
    pltpu.sync_copy(zero_v, acc_sh.at[pl.ds(s * DEG_PER_TILE, DEG_PER_TILE)])
    pltpu.sync_copy(ones_hbm, ones_v)
    ibase = c * (IDX_ROWS // NC) + s * DEG_IDX_ROWS
    pltpu.sync_copy(dst_hbm.at[pl.ds(ibase, DEG_IDX_ROWS)], idx_v)
    plsc.subcore_barrier()

    def body(i, carry):
        pltpu.sync_copy(ones_v, acc_sh.at[idx_v.at[i]], add=True)
        return carry

    lax.fori_loop(0, DEG_IDX_ROWS, body, 0)
    plsc.subcore_barrier()
    pltpu.sync_copy(
        acc_sh.at[pl.ds(s * DEG_PER_TILE, DEG_PER_TILE)],
        out_hbm.at[c, pl.ds(s * DEG_PER_TILE, DEG_PER_TILE)],
    )


@functools.partial(
    pl.kernel,
    out_type=jax.ShapeDtypeStruct((NC, NP, HALF), jnp.float32),
    mesh=_MESH,
    scratch_types=[
        pltpu.VMEM((IDX_BLOCK, CHUNK), jnp.int32),
        pltpu.VMEM((IDX_BLOCK, CHUNK), jnp.int32),
        pltpu.VMEM((CHUNK, HALF), jnp.float32),
        pltpu.VMEM((CHUNK, HALF), jnp.float32),
        pltpu.VMEM_SHARED((NP, HALF), jnp.float32),
        pltpu.SemaphoreType.DMA,
        pltpu.SemaphoreType.DMA,
    ],
)
def _sc_aggregate(hp_hbm, srcab_hbm, dst_hbm, zeros_hbm, out_hbm,
                  sidx_v, didx_v, rows0_v, rows1_v, acc_sh, sem0, sem1):
    c = lax.axis_index("c")
    s = lax.axis_index("s")
    rbase = s * ROWS_PER_TILE
    pltpu.sync_copy(
        zeros_hbm.at[pl.ds(rbase, ROWS_PER_TILE)],
        acc_sh.at[pl.ds(rbase, ROWS_PER_TILE)],
    )
    ibase = s * TILE_IDX_ROWS
    plsc.subcore_barrier()

    def block(b, carry):
        pltpu.sync_copy(
            srcab_hbm.at[c, pl.ds(ibase + b * IDX_BLOCK, IDX_BLOCK)], sidx_v
        )
        pltpu.sync_copy(
            dst_hbm.at[pl.ds(ibase + b * IDX_BLOCK, IDX_BLOCK)], didx_v
        )
        pltpu.async_copy(hp_hbm.at[sidx_v.at[0]], rows0_v, sem0)

        def pair(k, c2):
            i0 = 2 * k
            i1 = i0 + 1
            pltpu.make_async_copy(
                zeros_hbm.at[pl.ds(0, CHUNK)], rows0_v, sem0).wait()
            pltpu.async_copy(hp_hbm.at[sidx_v.at[i1]], rows1_v, sem1)
            pltpu.sync_copy(rows0_v, acc_sh.at[didx_v.at[i0]], add=True)
            pltpu.make_async_copy(
                zeros_hbm.at[pl.ds(0, CHUNK)], rows1_v, sem1).wait()

            @pl.when(k < IDX_BLOCK // 2 - 1)
            def _():
                pltpu.async_copy(hp_hbm.at[sidx_v.at[i0 + 2]], rows0_v, sem0)

            pltpu.sync_copy(rows1_v, acc_sh.at[didx_v.at[i1]], add=True)
            return c2

        lax.fori_loop(0, IDX_BLOCK // 2, pair, 0)
        return carry

    lax.fori_loop(0, TILE_IDX_ROWS // IDX_BLOCK, block, 0)
    plsc.subcore_barrier()
    pltpu.sync_copy(
        acc_sh.at[pl.ds(rbase, ROWS_PER_TILE)],
        out_hbm.at[c, pl.ds(rbase, ROWS_PER_TILE)],
    )


def _tc1_body(x_ref, w_ref, dis_ref, out_ref):
    h = jnp.dot(x_ref[...], w_ref[...], preferred_element_type=jnp.float32)
    hp = dis_ref[...] * h
    out_ref[0] = hp[:, :HALF]
    out_ref[1] = hp[:, HALF:]


def _tc_mid_body(aa_ref, ha_ref, dis_ref, b_ref, w_ref, out_ref):
    a = jnp.concatenate([aa_ref[0] + ha_ref[0], aa_ref[1] + ha_ref[1]], axis=1)
    d = dis_ref[...]
    h = jnp.maximum(d * a + b_ref[0:1, :], 0.0)
    o = d * jnp.dot(h, w_ref[...], preferred_element_type=jnp.float32)
    out_ref[0] = o[:, :HALF]
    out_ref[1] = o[:, HALF:]


def _tc_final_body(aa_ref, ha_ref, dis_ref, b_ref, wl_ref, bl_ref, out_ref):
    a = jnp.concatenate([aa_ref[0] + ha_ref[0], aa_ref[1] + ha_ref[1]], axis=1)
    d = dis_ref[...]
    h = jnp.maximum(d * a + b_ref[0:1, :], 0.0)
    z = jnp.dot(h, wl_ref[...], preferred_element_type=jnp.float32)
    z = z + bl_ref[0:1, 0:1]
    out_ref[...] = 1.0 / (1.0 + jnp.exp(-z))


_R = 5120
_G = NP // _R


def kernel(x, edge_index, W1, b1, W2, b2, Wl, bl):
    src = edge_index[0].astype(jnp.int32)
    dst = edge_index[1].astype(jnp.int32)

    pad_n = E_PAD - N_EDGES
    pad_src = jnp.arange(pad_n, dtype=jnp.int32) % N_NODES
    pad_dst = N_NODES + (jnp.arange(pad_n, dtype=jnp.int32) % (NP - N_NODES))
    src_p = jnp.concatenate([src, pad_src])
    dst_p = jnp.concatenate([dst, pad_dst])
    src_ab = jnp.stack([src_p, src_p + NP]).reshape(NC, IDX_ROWS, CHUNK)
    dst2d = dst_p.reshape(IDX_ROWS, CHUNK)

    xp = jnp.pad(x, ((0, NP - N_NODES), (0, 0)))
    ones_hbm = jnp.ones((CHUNK,), jnp.float32)

    partials = _sc_degree(dst2d, ones_hbm)
    deg = partials[0] + partials[1] + 1.0
    dis = lax.rsqrt(deg).reshape(NP, 1)

    zeros = jnp.zeros((NP, HALF), jnp.float32)
    b1b = jnp.broadcast_to(b1.reshape(1, HID), (8, HID))
    b2b = jnp.broadcast_to(b2.reshape(1, HID), (8, HID))
    wlp = jnp.pad(Wl, ((0, 0), (0, HALF - 1)))
    blb = jnp.broadcast_to(bl.reshape(1, 1), (8, HALF))

    hp1 = pl.pallas_call(
        _tc1_body,
        grid=(_G,),
        in_specs=[
            pl.BlockSpec((_R, IN_CH), lambda i: (i, 0)),
            pl.BlockSpec((IN_CH, HID), lambda i: (0, 0)),
            pl.BlockSpec((_R, 1), lambda i: (i, 0)),
        ],
        out_specs=pl.BlockSpec((2, _R, HALF), lambda i: (0, i, 0)),
        out_shape=jax.ShapeDtypeStruct((2, NP, HALF), jnp.float32),
    )(xp, W1, dis)

    agg1 = _sc_aggregate(hp1.reshape(2 * NP, HALF), src_ab, dst2d, zeros)

    hp2 = pl.pallas_call(
        _tc_mid_body,
        grid=(_G,),
        in_specs=[
            pl.BlockSpec((2, _R, HALF), lambda i: (0, i, 0)),
            pl.BlockSpec((2, _R, HALF), lambda i: (0, i, 0)),
            pl.BlockSpec((_R, 1), lambda i: (i, 0)),
            pl.BlockSpec((8, HID), lambda i: (0, 0)),
            pl.BlockSpec((HID, HID), lambda i: (0, 0)),
        ],
        out_specs=pl.BlockSpec((2, _R, HALF), lambda i: (0, i, 0)),
        out_shape=jax.ShapeDtypeStruct((2, NP, HALF), jnp.float32),
    )(agg1, hp1, dis, b1b, W2)

    agg2 = _sc_aggregate(hp2.reshape(2 * NP, HALF), src_ab, dst2d, zeros)

    o = pl.pallas_call(
        _tc_final_body,
        grid=(_G,),
        in_specs=[
            pl.BlockSpec((2, _R, HALF), lambda i: (0, i, 0)),
            pl.BlockSpec((2, _R, HALF), lambda i: (0, i, 0)),
            pl.BlockSpec((_R, 1), lambda i: (i, 0)),
            pl.BlockSpec((8, HID), lambda i: (0, 0)),
            pl.BlockSpec((HID, HALF), lambda i: (0, 0)),
            pl.BlockSpec((8, HALF), lambda i: (0, 0)),
        ],
        out_specs=pl.BlockSpec((_R, HALF), lambda i: (i, 0)),
        out_shape=jax.ShapeDtypeStruct((NP, HALF), jnp.float32),
    )(agg2, hp2, dis, b2b, wlp, blb)

    return o[:N_NODES, 0]
